# Initial kernel scaffold; baseline (speedup 1.0000x reference)
import jax
import jax.numpy as jnp
from jax.experimental import pallas as pl


def kernel(x, edge_index):
    raise NotImplementedError("write your pallas kernel here")


# speedup vs baseline: 4.6478x; 4.6478x over previous
import functools

import jax
import jax.numpy as jnp
from jax import lax
from jax.experimental import pallas as pl
from jax.experimental.pallas import tpu as pltpu
from jax.experimental.pallas import tpu_sc as plsc

N = 10000
D = 128
E = 320000

NC = 2
NS = 16
NW = NC * NS

C = 128
CHUNKS = 79
EPT = C * CHUNKS
EPAD = EPT * NW

NPAD = 10240
RPT = NPAD // NS

DEGW = 16

_mesh = plsc.VectorSubcoreMesh(core_axis_name="c", subcore_axis_name="s")


SLICE = NPAD // NS


@functools.partial(
    pl.kernel,
    out_type=jax.ShapeDtypeStruct((NC * NPAD,), jnp.float32),
    mesh=_mesh,
    compiler_params=pltpu.CompilerParams(needs_layout_passes=False),
    scratch_types=[
        pltpu.VMEM((EPT,), jnp.int32),
        pltpu.VMEM((NPAD,), jnp.float32),
        pltpu.VMEM((NS, SLICE), jnp.float32),
        pltpu.VMEM((SLICE,), jnp.float32),
        pltpu.VMEM_SHARED((NS, NPAD), jnp.float32),
    ],
)
def _deg_kernel(dst_hbm, out_hbm, didx_all, hist, partbuf, result, acc):
    c = lax.axis_index("c")
    s = lax.axis_index("s")
    wid = c * NS + s

    zrow = jnp.zeros((16,), jnp.float32)
    for g in range(NPAD // 16):
        hist[pl.ds(g * 16, 16)] = zrow

    pltpu.sync_copy(dst_hbm.at[pl.ds(wid * EPT, EPT)], didx_all)
    one16 = jnp.ones((16,), jnp.float32)

    def body(g, _):
        v = didx_all[pl.ds(g * 16, 16)]
        plsc.addupdate_scatter(hist, [v], one16)
        return ()

    lax.fori_loop(0, EPT // 16, body, ())

    pltpu.sync_copy(hist, acc.at[s])
    plsc.subcore_barrier()

    for p in range(NS):
        pltpu.sync_copy(acc.at[p, pl.ds(s * SLICE, SLICE)], partbuf.at[p])

    def merge(g, _):
        tot = partbuf[0, pl.ds(g * 16, 16)]
        for p in range(1, NS):
            tot = tot + partbuf[p, pl.ds(g * 16, 16)]
        result[pl.ds(g * 16, 16)] = tot
        return ()

    lax.fori_loop(0, SLICE // 16, merge, ())
    pltpu.sync_copy(result, out_hbm.at[pl.ds(c * NPAD + s * SLICE, SLICE)])


@functools.partial(
    pl.kernel,
    out_type=jax.ShapeDtypeStruct((NC * NPAD, D), jnp.float32),
    mesh=_mesh,
    scratch_types=[
        pltpu.VMEM((C,), jnp.int32),
        pltpu.VMEM((C,), jnp.int32),
        pltpu.VMEM((C, D), jnp.float32),
        pltpu.VMEM((40, D), jnp.float32),
        pltpu.VMEM_SHARED((NPAD, D), jnp.float32),
        pltpu.SemaphoreType.DMA,
    ],
)
def _agg_kernel(src_hbm, dst_hbm, hsrc_hbm, out_hbm,
                sidx, didx, rows_v, zbuf, acc, sem):
    c = lax.axis_index("c")
    s = lax.axis_index("s")
    wid = c * NS + s

    zrow = jnp.zeros((16,), jnp.float32)
    for r in range(40):
        for k in range(D // 16):
            zbuf[r, pl.ds(k * 16, 16)] = zrow

    row0 = s * RPT
    for j in range(RPT // 40):
        pltpu.sync_copy(zbuf, acc.at[pl.ds(row0 + j * 40, 40)])
    plsc.subcore_barrier()

    ebase = wid * EPT

    def body(j, _):
        e0 = ebase + j * C
        pltpu.sync_copy(src_hbm.at[pl.ds(e0, C)], sidx)
        pltpu.sync_copy(dst_hbm.at[pl.ds(e0, C)], didx)
        pltpu.async_copy(hsrc_hbm.at[sidx], rows_v, sem).wait()
        pltpu.sync_copy(rows_v, acc.at[didx], add=True)
        return ()

    lax.fori_loop(0, CHUNKS, body, ())
    plsc.subcore_barrier()

    pltpu.sync_copy(acc.at[pl.ds(row0, RPT)],
                    out_hbm.at[pl.ds(c * NPAD + row0, RPT)])


def _scale_body(deg0_ref, deg1_ref, x_ref, o_ref):
    d = deg0_ref[...] + deg1_ref[...]
    o_ref[...] = x_ref[...] * lax.rsqrt(jnp.maximum(d, 1.0))


BLK = 1024


def _scale_call(deg0, deg1, x_pad):
    nb = NPAD // BLK
    return pl.pallas_call(
        _scale_body,
        grid=(nb,),
        in_specs=[
            pl.BlockSpec((BLK, 1), lambda i: (i, 0)),
            pl.BlockSpec((BLK, 1), lambda i: (i, 0)),
            pl.BlockSpec((BLK, D), lambda i: (i, 0)),
        ],
        out_specs=pl.BlockSpec((BLK, D), lambda i: (i, 0)),
        out_shape=jax.ShapeDtypeStruct((NPAD, D), jnp.float32),
    )(deg0, deg1, x_pad)


def _final_body(deg0_ref, deg1_ref, a0_ref, a1_ref, x_ref, o_ref):
    d = deg0_ref[...] + deg1_ref[...]
    agg = a0_ref[...] + a1_ref[...]
    o_ref[...] = x_ref[...] - agg * lax.rsqrt(jnp.maximum(d, 1.0))


def _final_call(deg0, deg1, agg, x_pad):
    nbp = NPAD // BLK
    nb = nbp
    return pl.pallas_call(
        _final_body,
        grid=(nb,),
        in_specs=[
            pl.BlockSpec((BLK, 1), lambda i: (i, 0)),
            pl.BlockSpec((BLK, 1), lambda i: (i, 0)),
            pl.BlockSpec((BLK, D), lambda i: (i, 0)),
            pl.BlockSpec((BLK, D), lambda i: (i + nbp, 0)),
            pl.BlockSpec((BLK, D), lambda i: (i, 0)),
        ],
        out_specs=pl.BlockSpec((BLK, D), lambda i: (i, 0)),
        out_shape=jax.ShapeDtypeStruct((N, D), jnp.float32),
    )(deg0, deg1, agg, agg, x_pad)


def kernel(x, edge_index):
    src = edge_index[0].astype(jnp.int32)
    dst = edge_index[1].astype(jnp.int32)
    pad = jnp.full((EPAD - E,), N, jnp.int32)
    src_p = jnp.concatenate([src, pad])
    dst_p = jnp.concatenate([dst, pad])
    x_pad = jnp.zeros((NPAD, D), jnp.float32).at[:N].set(x)

    deg = _deg_kernel(dst_p)
    deg0 = deg[:NPAD].reshape(NPAD, 1)
    deg1 = deg[NPAD:].reshape(NPAD, 1)
    h_src = _scale_call(deg0, deg1, x_pad)
    agg = _agg_kernel(src_p, dst_p, h_src)
    return _final_call(deg0, deg1, agg, x_pad)
